# baseline (speedup 1.0000x reference)
import jax
import jax.numpy as jnp
from jax.experimental import pallas as pl
from jax.experimental.pallas import tpu as pltpu

_TOTAL = 2048 * 5023 * 3
_LANES = 128
_ROWS = _TOTAL // _LANES
_BLOCK_ROWS = 4096
_GRID = (_ROWS + _BLOCK_ROWS - 1) // _BLOCK_ROWS
_TAIL_VALID = _ROWS - (_GRID - 1) * _BLOCK_ROWS


def _mse_kernel(p_ref, r_ref, out_ref, acc_ref):
    i = pl.program_id(0)

    @pl.when(i == 0)
    def _init():
        acc_ref[...] = jnp.zeros_like(acc_ref)

    d = p_ref[...] - r_ref[...]
    dsq = d * d

    @pl.when(i < _GRID - 1)
    def _body():
        acc_ref[...] += jnp.sum(
            dsq.reshape(_BLOCK_ROWS // 8, 8, _LANES), axis=0)

    @pl.when(i == _GRID - 1)
    def _tail():
        row = jax.lax.broadcasted_iota(jnp.int32, (_BLOCK_ROWS, _LANES), 0)
        masked = jnp.where(row < _TAIL_VALID, dsq, 0.0)
        acc = acc_ref[...] + jnp.sum(
            masked.reshape(_BLOCK_ROWS // 8, 8, _LANES), axis=0)
        out_ref[0] = jnp.sum(acc)


def kernel(pred_vertices, ref_vertices):
    p = pred_vertices.reshape(_ROWS, _LANES)
    r = ref_vertices.reshape(_ROWS, _LANES)
    total = pl.pallas_call(
        _mse_kernel,
        grid=(_GRID,),
        in_specs=[
            pl.BlockSpec((_BLOCK_ROWS, _LANES), lambda i: (i, 0)),
            pl.BlockSpec((_BLOCK_ROWS, _LANES), lambda i: (i, 0)),
        ],
        out_specs=pl.BlockSpec(memory_space=pltpu.MemorySpace.SMEM),
        out_shape=jax.ShapeDtypeStruct((1,), jnp.float32),
        scratch_shapes=[pltpu.VMEM((8, _LANES), jnp.float32)],
    )(p, r)
    return (total[0] / _TOTAL).astype(jnp.float32)


# speedup vs baseline: 947.0609x; 947.0609x over previous
import jax
import jax.numpy as jnp
from jax.experimental import pallas as pl
from jax.experimental.pallas import tpu as pltpu

_D0 = 3
_D1 = 5023
_D2 = 2048
_TOTAL = _D0 * _D1 * _D2
_BS = 128
_GRID = (_D1 + _BS - 1) // _BS
_TAIL_VALID = _D1 - (_GRID - 1) * _BS


def _mse_kernel(p_ref, r_ref, out_ref, acc_ref):
    i = pl.program_id(0)

    @pl.when(i == 0)
    def _init():
        acc_ref[...] = jnp.zeros_like(acc_ref)

    d = p_ref[...] - r_ref[...]
    dsq = d * d

    @pl.when(i < _GRID - 1)
    def _body():
        acc_ref[...] += jnp.sum(
            dsq.reshape(_D0 * _BS // 8, 8, _D2), axis=0)

    @pl.when(i == _GRID - 1)
    def _tail():
        row = jax.lax.broadcasted_iota(jnp.int32, (_D0, _BS, _D2), 1)
        masked = jnp.where(row < _TAIL_VALID, dsq, 0.0)
        acc = acc_ref[...] + jnp.sum(
            masked.reshape(_D0 * _BS // 8, 8, _D2), axis=0)
        out_ref[0] = jnp.sum(acc)


def kernel(pred_vertices, ref_vertices):
    p = jnp.transpose(pred_vertices, (2, 1, 0))
    r = jnp.transpose(ref_vertices, (2, 1, 0))
    total = pl.pallas_call(
        _mse_kernel,
        grid=(_GRID,),
        in_specs=[
            pl.BlockSpec((_D0, _BS, _D2), lambda i: (0, i, 0)),
            pl.BlockSpec((_D0, _BS, _D2), lambda i: (0, i, 0)),
        ],
        out_specs=pl.BlockSpec(memory_space=pltpu.MemorySpace.SMEM),
        out_shape=jax.ShapeDtypeStruct((1,), jnp.float32),
        scratch_shapes=[pltpu.VMEM((8, _D2), jnp.float32)],
    )(p, r)
    return (total[0] / _TOTAL).astype(jnp.float32)
